# Initial kernel scaffold; baseline (speedup 1.0000x reference)
import functools

import jax
import jax.numpy as jnp
from jax import lax
from jax.experimental import pallas as pl
from jax.experimental.pallas import tpu as pltpu
from jax.experimental.pallas import tpu_sc as plsc

NC = 2
NS = 16
CHUNK = 125


def _deg_kernel_body(dst4, ones_hbm, zeros_hbm, deg_out, dstv, onesv, degsp):
    c = lax.axis_index("c")
    s = lax.axis_index("s")
    wid = c * NS + s
    n_chunks = dst4.shape[1]
    rows_per_tile = degsp.shape[0] // NS

    pltpu.sync_copy(zeros_hbm, degsp.at[pl.ds(s * rows_per_tile, rows_per_tile)])
    pltpu.sync_copy(dst4.at[wid], dstv)
    pltpu.sync_copy(ones_hbm, onesv)
    plsc.subcore_barrier()

    def body(j, carry):
        pltpu.sync_copy(onesv, degsp.at[dstv.at[j]], add=True)
        return carry

    lax.fori_loop(0, n_chunks, body, 0)
    plsc.subcore_barrier()
    sl = pl.ds(s * rows_per_tile, rows_per_tile)
    pltpu.sync_copy(degsp.at[sl], deg_out.at[c, sl])


def _deg_call(dst4, n):
    mesh = plsc.VectorSubcoreMesh(core_axis_name="c", subcore_axis_name="s")
    n_chunks = dst4.shape[1]
    rows_per_tile = n // NS
    k = pl.kernel(
        _deg_kernel_body,
        out_type=jax.ShapeDtypeStruct((NC, n, 16), jnp.float32),
        mesh=mesh,
        scratch_types=[
            pltpu.VMEM((n_chunks, CHUNK), jnp.int32),
            pltpu.VMEM((CHUNK, 16), jnp.float32),
            pltpu.VMEM_SHARED((n, 16), jnp.float32),
        ],
    )
    ones_hbm = jnp.ones((CHUNK, 16), jnp.float32)
    zeros_hbm = jnp.zeros((rows_per_tile, 16), jnp.float32)
    return k(dst4, ones_hbm, zeros_hbm)


def _agg_kernel_body(hs_a, hs_b, src3, dst3, zeros_hbm, g_a, g_b,
                     srcv, dstv, rows, accsp, sem):
    c = lax.axis_index("c")
    s = lax.axis_index("s")
    n_chunks = src3.shape[1]
    n = accsp.shape[0]
    rows_per_tile = n // NS
    sl = pl.ds(s * rows_per_tile, rows_per_tile)

    pltpu.sync_copy(zeros_hbm, accsp.at[sl])
    pltpu.sync_copy(src3.at[s], srcv)
    pltpu.sync_copy(dst3.at[s], dstv)
    plsc.subcore_barrier()

    def body(j, carry):
        @pl.when(c == 0)
        def _():
            pltpu.async_copy(hs_a.at[srcv.at[j]], rows, sem).wait()

        @pl.when(c == 1)
        def _():
            pltpu.async_copy(hs_b.at[srcv.at[j]], rows, sem).wait()

        pltpu.sync_copy(rows, accsp.at[dstv.at[j]], add=True)
        return carry

    lax.fori_loop(0, n_chunks, body, 0)
    plsc.subcore_barrier()

    @pl.when(c == 0)
    def _():
        pltpu.sync_copy(accsp.at[sl], g_a.at[sl])

    @pl.when(c == 1)
    def _():
        pltpu.sync_copy(accsp.at[sl], g_b.at[sl])


def _agg_call(hs_a, hs_b, src3, dst3):
    n, h = hs_a.shape
    n_chunks = src3.shape[1]
    mesh = plsc.VectorSubcoreMesh(core_axis_name="c", subcore_axis_name="s")
    k = pl.kernel(
        _agg_kernel_body,
        out_type=(
            jax.ShapeDtypeStruct((n, h), jnp.float32),
            jax.ShapeDtypeStruct((n, h), jnp.float32),
        ),
        mesh=mesh,
        scratch_types=[
            pltpu.VMEM((n_chunks, CHUNK), jnp.int32),
            pltpu.VMEM((n_chunks, CHUNK), jnp.int32),
            pltpu.VMEM((CHUNK, h), jnp.float32),
            pltpu.VMEM_SHARED((n, h), jnp.float32),
            pltpu.SemaphoreType.DMA,
        ],
    )
    zeros_hbm = jnp.zeros((n // NS, h), jnp.float32)
    return k(hs_a, hs_b, src3, dst3, zeros_hbm)


_BR = 500


def _mm1_body(x_ref, w_ref, deg_ref, sa_ref, sb_ref, dinv_ref):
    deg = deg_ref[0, :, 0] + deg_ref[1, :, 0] + 1.0
    dinv = lax.rsqrt(deg)
    h = jnp.dot(x_ref[...], w_ref[...], preferred_element_type=jnp.float32)
    s = h * dinv[:, None]
    half = s.shape[1] // 2
    sa_ref[...] = s[:, :half]
    sb_ref[...] = s[:, half:]
    dinv_ref[...] = dinv[:, None]


def _mm1_call(x, w, deg_out):
    n, d_in = x.shape
    d = w.shape[1]
    half = d // 2
    grid = (n // _BR,)
    return pl.pallas_call(
        _mm1_body,
        grid=grid,
        in_specs=[
            pl.BlockSpec((_BR, d_in), lambda i: (i, 0)),
            pl.BlockSpec((d_in, d), lambda i: (0, 0)),
            pl.BlockSpec((NC, _BR, 16), lambda i: (0, i, 0)),
        ],
        out_specs=[
            pl.BlockSpec((_BR, half), lambda i: (i, 0)),
            pl.BlockSpec((_BR, half), lambda i: (i, 0)),
            pl.BlockSpec((_BR, 1), lambda i: (i, 0)),
        ],
        out_shape=[
            jax.ShapeDtypeStruct((n, half), jnp.float32),
            jax.ShapeDtypeStruct((n, half), jnp.float32),
            jax.ShapeDtypeStruct((n, 1), jnp.float32),
        ],
    )(x, w, deg_out)


def _mm2_body(ga_ref, gb_ref, sa_ref, sb_ref, dinv_ref, b_ref, w_ref,
              oa_ref, ob_ref):
    dinv = dinv_ref[...]
    half = ga_ref.shape[1]
    b = b_ref[...]
    xa = jnp.maximum(dinv * (ga_ref[...] + sa_ref[...]) + b[:, :half], 0.0)
    xb = jnp.maximum(dinv * (gb_ref[...] + sb_ref[...]) + b[:, half:], 0.0)
    x1 = jnp.concatenate([xa, xb], axis=1)
    t2 = jnp.dot(x1, w_ref[...], preferred_element_type=jnp.float32)
    s2 = t2 * dinv
    h2 = s2.shape[1] // 2
    oa_ref[...] = s2[:, :h2]
    ob_ref[...] = s2[:, h2:]


def _mm2_call(ga, gb, sa, sb, dinv, b1, w2):
    n, half = ga.shape
    d_out = w2.shape[1]
    h2 = d_out // 2
    grid = (n // _BR,)
    return pl.pallas_call(
        _mm2_body,
        grid=grid,
        in_specs=[
            pl.BlockSpec((_BR, half), lambda i: (i, 0)),
            pl.BlockSpec((_BR, half), lambda i: (i, 0)),
            pl.BlockSpec((_BR, half), lambda i: (i, 0)),
            pl.BlockSpec((_BR, half), lambda i: (i, 0)),
            pl.BlockSpec((_BR, 1), lambda i: (i, 0)),
            pl.BlockSpec((1, 2 * half), lambda i: (0, 0)),
            pl.BlockSpec((2 * half, d_out), lambda i: (0, 0)),
        ],
        out_specs=[
            pl.BlockSpec((_BR, h2), lambda i: (i, 0)),
            pl.BlockSpec((_BR, h2), lambda i: (i, 0)),
        ],
        out_shape=[
            jax.ShapeDtypeStruct((n, h2), jnp.float32),
            jax.ShapeDtypeStruct((n, h2), jnp.float32),
        ],
    )(ga, gb, sa, sb, dinv, b1, w2)


def _final_body(ga_ref, gb_ref, sa_ref, sb_ref, dinv_ref, b_ref, o_ref):
    dinv = dinv_ref[...]
    b = b_ref[...]
    h2 = ga_ref.shape[1]
    oa = dinv * (ga_ref[...] + sa_ref[...]) + b[:, :h2]
    ob = dinv * (gb_ref[...] + sb_ref[...]) + b[:, h2:]
    o_ref[...] = jnp.concatenate([oa, ob], axis=1)


def _final_call(ga, gb, sa, sb, dinv, b2):
    n, h2 = ga.shape
    d_out = 2 * h2
    grid = (n // _BR,)
    return pl.pallas_call(
        _final_body,
        grid=grid,
        in_specs=[
            pl.BlockSpec((_BR, h2), lambda i: (i, 0)),
            pl.BlockSpec((_BR, h2), lambda i: (i, 0)),
            pl.BlockSpec((_BR, h2), lambda i: (i, 0)),
            pl.BlockSpec((_BR, h2), lambda i: (i, 0)),
            pl.BlockSpec((_BR, 1), lambda i: (i, 0)),
            pl.BlockSpec((1, d_out), lambda i: (0, 0)),
        ],
        out_specs=pl.BlockSpec((_BR, d_out), lambda i: (i, 0)),
        out_shape=jax.ShapeDtypeStruct((n, d_out), jnp.float32),
    )(ga, gb, sa, sb, dinv, b2)


@jax.jit
def kernel(X, A, W1, b1, W2, b2):
    n = X.shape[0]
    e = A.shape[1]
    src = A[0]
    dst = A[1]
    epp = e // NS
    src3 = src.reshape(NS, epp // CHUNK, CHUNK)
    dst3 = dst.reshape(NS, epp // CHUNK, CHUNK)
    dst4 = dst.reshape(NC * NS, e // (NC * NS) // CHUNK, CHUNK)

    deg_out = _deg_call(dst4, n)
    s1a, s1b, dinv = _mm1_call(X, W1, deg_out)
    g1a, g1b = _agg_call(s1a, s1b, src3, dst3)
    s2a, s2b = _mm2_call(g1a, g1b, s1a, s1b, dinv, b1.reshape(1, -1), W2)
    g2a, g2b = _agg_call(s2a, s2b, src3, dst3)
    return _final_call(g2a, g2b, s2a, s2b, dinv, b2.reshape(1, -1))


# speedup vs baseline: 13.0193x; 13.0193x over previous
import jax
import jax.numpy as jnp
from jax import lax
from jax.experimental import pallas as pl
from jax.experimental.pallas import tpu as pltpu
from jax.experimental.pallas import tpu_sc as plsc

NC = 2
NS = 16
CHUNK = 125


def _span(n):
    stride = ((n // NS) // 8) * 8
    return stride, n - stride * (NS - 1)


def _deg_kernel_body(dst4, ones_hbm, zeros_hbm, deg_out, dstv, onesv, degsp):
    c = lax.axis_index("c")
    s = lax.axis_index("s")
    wid = c * NS + s
    n_chunks = dst4.shape[1]
    n = degsp.shape[0]
    stride, span = _span(n)
    sl = pl.ds(s * stride, span)

    pltpu.sync_copy(zeros_hbm, degsp.at[sl])
    pltpu.sync_copy(dst4.at[wid], dstv)
    pltpu.sync_copy(ones_hbm, onesv)
    plsc.subcore_barrier()

    def body(j, carry):
        pltpu.sync_copy(onesv, degsp.at[dstv.at[j]], add=True)
        return carry

    lax.fori_loop(0, n_chunks, body, 0)
    plsc.subcore_barrier()
    pltpu.sync_copy(degsp.at[sl], deg_out.at[c, sl])


def _deg_call(dst4, n):
    mesh = plsc.VectorSubcoreMesh(core_axis_name="c", subcore_axis_name="s")
    n_chunks = dst4.shape[1]
    _, span = _span(n)
    k = pl.kernel(
        _deg_kernel_body,
        out_type=jax.ShapeDtypeStruct((NC, n, 128), jnp.float32),
        mesh=mesh,
        scratch_types=[
            pltpu.VMEM((n_chunks, CHUNK), jnp.int32),
            pltpu.VMEM((CHUNK, 128), jnp.float32),
            pltpu.VMEM_SHARED((n, 128), jnp.float32),
        ],
    )
    ones_hbm = jnp.ones((CHUNK, 128), jnp.float32)
    zeros_hbm = jnp.zeros((span, 128), jnp.float32)
    return k(dst4, ones_hbm, zeros_hbm)


def _agg1_kernel_body(hs_a, hs_b, src3, dst3, zeros_hbm, g_a, g_b,
                      srcv, dstv, rows, accsp, sem):
    c = lax.axis_index("c")
    s = lax.axis_index("s")
    n_chunks = src3.shape[1]
    n = accsp.shape[0]
    stride, span = _span(n)
    sl = pl.ds(s * stride, span)

    pltpu.sync_copy(zeros_hbm, accsp.at[sl])
    pltpu.sync_copy(src3.at[s], srcv)
    pltpu.sync_copy(dst3.at[s], dstv)
    plsc.subcore_barrier()

    def body(j, carry):
        @pl.when(c == 0)
        def _():
            pltpu.async_copy(hs_a.at[srcv.at[j]], rows, sem).wait()

        @pl.when(c == 1)
        def _():
            pltpu.async_copy(hs_b.at[srcv.at[j]], rows, sem).wait()

        pltpu.sync_copy(rows, accsp.at[dstv.at[j]], add=True)
        return carry

    lax.fori_loop(0, n_chunks, body, 0)
    plsc.subcore_barrier()

    @pl.when(c == 0)
    def _():
        pltpu.sync_copy(accsp.at[sl], g_a.at[sl])

    @pl.when(c == 1)
    def _():
        pltpu.sync_copy(accsp.at[sl], g_b.at[sl])


def _agg1_call(hs_a, hs_b, src3, dst3):
    n, h = hs_a.shape
    n_chunks = src3.shape[1]
    mesh = plsc.VectorSubcoreMesh(core_axis_name="c", subcore_axis_name="s")
    k = pl.kernel(
        _agg1_kernel_body,
        out_type=(
            jax.ShapeDtypeStruct((n, h), jnp.float32),
            jax.ShapeDtypeStruct((n, h), jnp.float32),
        ),
        mesh=mesh,
        scratch_types=[
            pltpu.VMEM((n_chunks, CHUNK), jnp.int32),
            pltpu.VMEM((n_chunks, CHUNK), jnp.int32),
            pltpu.VMEM((CHUNK, h), jnp.float32),
            pltpu.VMEM_SHARED((n, h), jnp.float32),
            pltpu.SemaphoreType.DMA,
        ],
    )
    _, span = _span(n)
    zeros_hbm = jnp.zeros((span, h), jnp.float32)
    return k(hs_a, hs_b, src3, dst3, zeros_hbm)


def _agg2_kernel_body(hs, src4, dst4, zeros_hbm, g_part,
                      srcv, dstv, rows, accsp, sem):
    c = lax.axis_index("c")
    s = lax.axis_index("s")
    wid = c * NS + s
    n_chunks = src4.shape[1]
    n = accsp.shape[0]
    stride, span = _span(n)
    sl = pl.ds(s * stride, span)

    pltpu.sync_copy(zeros_hbm, accsp.at[sl])
    pltpu.sync_copy(src4.at[wid], srcv)
    pltpu.sync_copy(dst4.at[wid], dstv)
    plsc.subcore_barrier()

    def body(j, carry):
        pltpu.async_copy(hs.at[srcv.at[j]], rows, sem).wait()
        pltpu.sync_copy(rows, accsp.at[dstv.at[j]], add=True)
        return carry

    lax.fori_loop(0, n_chunks, body, 0)
    plsc.subcore_barrier()
    pltpu.sync_copy(accsp.at[sl], g_part.at[c, sl])


def _agg2_call(hs, src4, dst4):
    n, h = hs.shape
    n_chunks = src4.shape[1]
    mesh = plsc.VectorSubcoreMesh(core_axis_name="c", subcore_axis_name="s")
    k = pl.kernel(
        _agg2_kernel_body,
        out_type=jax.ShapeDtypeStruct((NC, n, h), jnp.float32),
        mesh=mesh,
        scratch_types=[
            pltpu.VMEM((n_chunks, CHUNK), jnp.int32),
            pltpu.VMEM((n_chunks, CHUNK), jnp.int32),
            pltpu.VMEM((CHUNK, h), jnp.float32),
            pltpu.VMEM_SHARED((n, h), jnp.float32),
            pltpu.SemaphoreType.DMA,
        ],
    )
    _, span = _span(n)
    zeros_hbm = jnp.zeros((span, h), jnp.float32)
    return k(hs, src4, dst4, zeros_hbm)


_BR = 400


def _mm1_body(x_ref, w_ref, deg_ref, sa_ref, sb_ref, dinv_ref):
    deg = deg_ref[0, :, 0] + deg_ref[1, :, 0] + 1.0
    dinv = lax.rsqrt(deg)
    h = jnp.dot(x_ref[...], w_ref[...], preferred_element_type=jnp.float32)
    s = h * dinv[:, None]
    half = s.shape[1] // 2
    sa_ref[...] = s[:, :half]
    sb_ref[...] = s[:, half:]
    dinv_ref[...] = dinv[:, None]


def _mm1_call(x, w, deg_out):
    n, d_in = x.shape
    d = w.shape[1]
    half = d // 2
    return pl.pallas_call(
        _mm1_body,
        grid=(n // _BR,),
        in_specs=[
            pl.BlockSpec((_BR, d_in), lambda i: (i, 0)),
            pl.BlockSpec((d_in, d), lambda i: (0, 0)),
            pl.BlockSpec((NC, _BR, 128), lambda i: (0, i, 0)),
        ],
        out_specs=[
            pl.BlockSpec((_BR, half), lambda i: (i, 0)),
            pl.BlockSpec((_BR, half), lambda i: (i, 0)),
            pl.BlockSpec((_BR, 1), lambda i: (i, 0)),
        ],
        out_shape=[
            jax.ShapeDtypeStruct((n, half), jnp.float32),
            jax.ShapeDtypeStruct((n, half), jnp.float32),
            jax.ShapeDtypeStruct((n, 1), jnp.float32),
        ],
    )(x, w, deg_out)


def _mm2_body(ga_ref, gb_ref, sa_ref, sb_ref, dinv_ref, b_ref, w_ref, o_ref):
    dinv = dinv_ref[...]
    half = ga_ref.shape[1]
    b = b_ref[...]
    xa = jnp.maximum(dinv * (ga_ref[...] + sa_ref[...]) + b[:, :half], 0.0)
    xb = jnp.maximum(dinv * (gb_ref[...] + sb_ref[...]) + b[:, half:], 0.0)
    x1 = jnp.concatenate([xa, xb], axis=1)
    t2 = jnp.dot(x1, w_ref[...], preferred_element_type=jnp.float32)
    o_ref[...] = t2 * dinv


def _mm2_call(ga, gb, sa, sb, dinv, b1, w2):
    n, half = ga.shape
    d_out = w2.shape[1]
    return pl.pallas_call(
        _mm2_body,
        grid=(n // _BR,),
        in_specs=[
            pl.BlockSpec((_BR, half), lambda i: (i, 0)),
            pl.BlockSpec((_BR, half), lambda i: (i, 0)),
            pl.BlockSpec((_BR, half), lambda i: (i, 0)),
            pl.BlockSpec((_BR, half), lambda i: (i, 0)),
            pl.BlockSpec((_BR, 1), lambda i: (i, 0)),
            pl.BlockSpec((1, 2 * half), lambda i: (0, 0)),
            pl.BlockSpec((2 * half, d_out), lambda i: (0, 0)),
        ],
        out_specs=pl.BlockSpec((_BR, d_out), lambda i: (i, 0)),
        out_shape=jax.ShapeDtypeStruct((n, d_out), jnp.float32),
    )(ga, gb, sa, sb, dinv, b1, w2)


def _final_body(gp_ref, s_ref, dinv_ref, b_ref, o_ref):
    dinv = dinv_ref[...]
    g = gp_ref[0] + gp_ref[1]
    o_ref[...] = dinv * (g + s_ref[...]) + b_ref[...]


def _final_call(g_part, s2, dinv, b2):
    _, n, d_out = g_part.shape
    return pl.pallas_call(
        _final_body,
        grid=(n // _BR,),
        in_specs=[
            pl.BlockSpec((NC, _BR, d_out), lambda i: (0, i, 0)),
            pl.BlockSpec((_BR, d_out), lambda i: (i, 0)),
            pl.BlockSpec((_BR, 1), lambda i: (i, 0)),
            pl.BlockSpec((1, d_out), lambda i: (0, 0)),
        ],
        out_specs=pl.BlockSpec((_BR, d_out), lambda i: (i, 0)),
        out_shape=jax.ShapeDtypeStruct((n, d_out), jnp.float32),
    )(g_part, s2, dinv, b2)


@jax.jit
def kernel(X, A, W1, b1, W2, b2):
    n = X.shape[0]
    e = A.shape[1]
    src = A[0]
    dst = A[1]
    src3 = src.reshape(NS, e // NS // CHUNK, CHUNK)
    dst3 = dst.reshape(NS, e // NS // CHUNK, CHUNK)
    src4 = src.reshape(NC * NS, e // (NC * NS) // CHUNK, CHUNK)
    dst4 = dst.reshape(NC * NS, e // (NC * NS) // CHUNK, CHUNK)

    deg_out = _deg_call(dst4, n)
    s1a, s1b, dinv = _mm1_call(X, W1, deg_out)
    g1a, g1b = _agg1_call(s1a, s1b, src3, dst3)
    s2 = _mm2_call(g1a, g1b, s1a, s1b, dinv, b1.reshape(1, -1), W2)
    g2_part = _agg2_call(s2, src4, dst4)
    return _final_call(g2_part, s2, dinv, b2.reshape(1, -1))


# speedup vs baseline: 17.4841x; 1.3429x over previous
import jax
import jax.numpy as jnp
from jax import lax
from jax.experimental import pallas as pl
from jax.experimental.pallas import tpu as pltpu
from jax.experimental.pallas import tpu_sc as plsc

NC = 2
NS = 16
CHUNK = 125


def _span(n):
    stride = ((n // NS) // 8) * 8
    return stride, n - stride * (NS - 1)


def _deg_kernel_body(dst4, ones_hbm, zeros_hbm, deg_out, dstv, onesv, degsp,
                     ss0, ss1):
    c = lax.axis_index("c")
    s = lax.axis_index("s")
    wid = c * NS + s
    n_chunks = dst4.shape[1]
    n = degsp.shape[0]
    stride, span = _span(n)
    sl = pl.ds(s * stride, span)
    ss = (ss0, ss1)

    pltpu.sync_copy(zeros_hbm, degsp.at[sl])
    pltpu.sync_copy(dst4.at[wid], dstv)
    pltpu.sync_copy(ones_hbm, onesv)
    plsc.subcore_barrier()

    def body(i, carry):
        for b in range(2):
            j = 2 * i + b

            @pl.when(j >= 2)
            def _():
                pltpu.make_async_copy(onesv, degsp.at[dstv.at[j - 2]],
                                      ss[b]).wait()

            pltpu.async_copy(onesv, degsp.at[dstv.at[j]], ss[b], add=True)
        return carry

    lax.fori_loop(0, n_chunks // 2, body, 0)
    pltpu.make_async_copy(onesv, degsp.at[dstv.at[n_chunks - 2]], ss[0]).wait()
    pltpu.make_async_copy(onesv, degsp.at[dstv.at[n_chunks - 1]], ss[1]).wait()
    plsc.subcore_barrier()
    pltpu.sync_copy(degsp.at[sl], deg_out.at[c, sl])


def _deg_call(dst4, n):
    mesh = plsc.VectorSubcoreMesh(core_axis_name="c", subcore_axis_name="s")
    n_chunks = dst4.shape[1]
    _, span = _span(n)
    k = pl.kernel(
        _deg_kernel_body,
        out_type=jax.ShapeDtypeStruct((NC, n, 128), jnp.float32),
        mesh=mesh,
        scratch_types=[
            pltpu.VMEM((n_chunks, CHUNK), jnp.int32),
            pltpu.VMEM((CHUNK, 128), jnp.float32),
            pltpu.VMEM_SHARED((n, 128), jnp.float32),
            pltpu.SemaphoreType.DMA,
            pltpu.SemaphoreType.DMA,
        ],
    )
    ones_hbm = jnp.ones((CHUNK, 128), jnp.float32)
    zeros_hbm = jnp.zeros((span, 128), jnp.float32)
    return k(dst4, ones_hbm, zeros_hbm)


def _make_agg_body(split_cols, n_chunks, blk):
    nblk = n_chunks // blk
    assert nblk % 2 == 0 and blk % 2 == 0

    def body(hs_a, hs_b, srci, dsti, zeros_hbm, *outs_and_scratch):
        if split_cols:
            (g_a, g_b, srcb0, srcb1, dstb0, dstb1, rows0, rows1, accsp,
             sg0, sg1, ss0, ss1, si0, si1) = outs_and_scratch
        else:
            (g_part, srcb0, srcb1, dstb0, dstb1, rows0, rows1, accsp,
             sg0, sg1, ss0, ss1, si0, si1) = outs_and_scratch
        c = lax.axis_index("c")
        s = lax.axis_index("s")
        tid = s if split_cols else c * NS + s
        n = accsp.shape[0]
        stride, span = _span(n)
        sl = pl.ds(s * stride, span)
        srcb = (srcb0, srcb1)
        dstb = (dstb0, dstb1)
        rows = (rows0, rows1)
        sg = (sg0, sg1)
        ss = (ss0, ss1)
        si = (si0, si1)

        def gather(idx_row, buf, sem):
            if split_cols:
                @pl.when(c == 0)
                def _():
                    pltpu.async_copy(hs_a.at[idx_row], buf, sem)

                @pl.when(c == 1)
                def _():
                    pltpu.async_copy(hs_b.at[idx_row], buf, sem)
            else:
                pltpu.async_copy(hs_a.at[idx_row], buf, sem)

        def gather_wait(idx_row, buf, sem):
            pltpu.make_async_copy(hs_a.at[idx_row], buf, sem).wait()

        def scatter_wait(buf, any_idx_row, sem):
            pltpu.make_async_copy(buf, accsp.at[any_idx_row], sem).wait()

        pltpu.sync_copy(srci.at[tid, pl.ds(0, blk)], srcb0)
        pltpu.sync_copy(dsti.at[tid, pl.ds(0, blk)], dstb0)
        gather(srcb0.at[0], rows0, sg0)
        pltpu.sync_copy(zeros_hbm, accsp.at[sl])
        plsc.subcore_barrier()

        def outer(i, carry):
            for pb in range(2):
                blkid = 2 * i + pb

                @pl.when(blkid + 1 < nblk)
                def _():
                    nxt = pl.ds((blkid + 1) * blk, blk)
                    pltpu.async_copy(srci.at[tid, nxt], srcb[1 - pb],
                                     si[1 - pb])
                    pltpu.async_copy(dsti.at[tid, nxt], dstb[1 - pb],
                                     si[1 - pb])

                for b in range(blk):
                    rb = b % 2
                    if b == 0:
                        @pl.when(blkid >= 1)
                        def _():
                            scatter_wait(rows[1 - rb], dstb[pb].at[0],
                                         ss[1 - rb])
                    else:
                        scatter_wait(rows[1 - rb], dstb[pb].at[b - 1],
                                     ss[1 - rb])
                    if b < blk - 1:
                        gather(srcb[pb].at[b + 1], rows[1 - rb], sg[1 - rb])
                    else:
                        @pl.when(blkid + 1 < nblk)
                        def _():
                            pltpu.make_async_copy(
                                srci.at[tid, pl.ds(0, blk)], srcb[1 - pb],
                                si[1 - pb]).wait()
                            pltpu.make_async_copy(
                                dsti.at[tid, pl.ds(0, blk)], dstb[1 - pb],
                                si[1 - pb]).wait()
                            gather(srcb[1 - pb].at[0], rows[1 - rb],
                                   sg[1 - rb])
                    gather_wait(srcb[pb].at[b], rows[rb], sg[rb])
                    pltpu.async_copy(rows[rb], accsp.at[dstb[pb].at[b]],
                                     ss[rb], add=True)
            return carry

        lax.fori_loop(0, nblk // 2, outer, 0)
        scatter_wait(rows[(blk - 1) % 2], dstb[(nblk - 1) % 2].at[blk - 1],
                     ss[(blk - 1) % 2])
        plsc.subcore_barrier()

        if split_cols:
            @pl.when(c == 0)
            def _():
                pltpu.sync_copy(accsp.at[sl], g_a.at[sl])

            @pl.when(c == 1)
            def _():
                pltpu.sync_copy(accsp.at[sl], g_b.at[sl])
        else:
            pltpu.sync_copy(accsp.at[sl], g_part.at[c, sl])

    return body


def _agg_call(split_cols, hs_a, hs_b, srci, dsti):
    n, h = hs_a.shape
    n_chunks = srci.shape[1]
    blk = 8 if n_chunks % 16 == 0 else 4
    mesh = plsc.VectorSubcoreMesh(core_axis_name="c", subcore_axis_name="s")
    if split_cols:
        out_type = (
            jax.ShapeDtypeStruct((n, h), jnp.float32),
            jax.ShapeDtypeStruct((n, h), jnp.float32),
        )
    else:
        out_type = jax.ShapeDtypeStruct((NC, n, h), jnp.float32)
    k = pl.kernel(
        _make_agg_body(split_cols, n_chunks, blk),
        out_type=out_type,
        mesh=mesh,
        scratch_types=[
            pltpu.VMEM((blk, CHUNK), jnp.int32),
            pltpu.VMEM((blk, CHUNK), jnp.int32),
            pltpu.VMEM((blk, CHUNK), jnp.int32),
            pltpu.VMEM((blk, CHUNK), jnp.int32),
            pltpu.VMEM((CHUNK, h), jnp.float32),
            pltpu.VMEM((CHUNK, h), jnp.float32),
            pltpu.VMEM_SHARED((n, h), jnp.float32),
            pltpu.SemaphoreType.DMA,
            pltpu.SemaphoreType.DMA,
            pltpu.SemaphoreType.DMA,
            pltpu.SemaphoreType.DMA,
            pltpu.SemaphoreType.DMA,
            pltpu.SemaphoreType.DMA,
        ],
    )
    _, span = _span(n)
    zeros_hbm = jnp.zeros((span, h), jnp.float32)
    return k(hs_a, hs_b, srci, dsti, zeros_hbm)


_BR = 400


def _mm1_body(x_ref, w_ref, deg_ref, sa_ref, sb_ref, dinv_ref):
    deg = deg_ref[0, :, 0] + deg_ref[1, :, 0] + 1.0
    dinv = lax.rsqrt(deg)
    h = jnp.dot(x_ref[...], w_ref[...], preferred_element_type=jnp.float32)
    s = h * dinv[:, None]
    half = s.shape[1] // 2
    sa_ref[...] = s[:, :half]
    sb_ref[...] = s[:, half:]
    dinv_ref[...] = dinv[:, None]


def _mm1_call(x, w, deg_out):
    n, d_in = x.shape
    d = w.shape[1]
    half = d // 2
    return pl.pallas_call(
        _mm1_body,
        grid=(n // _BR,),
        in_specs=[
            pl.BlockSpec((_BR, d_in), lambda i: (i, 0)),
            pl.BlockSpec((d_in, d), lambda i: (0, 0)),
            pl.BlockSpec((NC, _BR, 128), lambda i: (0, i, 0)),
        ],
        out_specs=[
            pl.BlockSpec((_BR, half), lambda i: (i, 0)),
            pl.BlockSpec((_BR, half), lambda i: (i, 0)),
            pl.BlockSpec((_BR, 1), lambda i: (i, 0)),
        ],
        out_shape=[
            jax.ShapeDtypeStruct((n, half), jnp.float32),
            jax.ShapeDtypeStruct((n, half), jnp.float32),
            jax.ShapeDtypeStruct((n, 1), jnp.float32),
        ],
    )(x, w, deg_out)


def _mm2_body(ga_ref, gb_ref, sa_ref, sb_ref, dinv_ref, b_ref, w_ref, o_ref):
    dinv = dinv_ref[...]
    half = ga_ref.shape[1]
    b = b_ref[...]
    xa = jnp.maximum(dinv * (ga_ref[...] + sa_ref[...]) + b[:, :half], 0.0)
    xb = jnp.maximum(dinv * (gb_ref[...] + sb_ref[...]) + b[:, half:], 0.0)
    x1 = jnp.concatenate([xa, xb], axis=1)
    t2 = jnp.dot(x1, w_ref[...], preferred_element_type=jnp.float32)
    o_ref[...] = t2 * dinv


def _mm2_call(ga, gb, sa, sb, dinv, b1, w2):
    n, half = ga.shape
    d_out = w2.shape[1]
    return pl.pallas_call(
        _mm2_body,
        grid=(n // _BR,),
        in_specs=[
            pl.BlockSpec((_BR, half), lambda i: (i, 0)),
            pl.BlockSpec((_BR, half), lambda i: (i, 0)),
            pl.BlockSpec((_BR, half), lambda i: (i, 0)),
            pl.BlockSpec((_BR, half), lambda i: (i, 0)),
            pl.BlockSpec((_BR, 1), lambda i: (i, 0)),
            pl.BlockSpec((1, 2 * half), lambda i: (0, 0)),
            pl.BlockSpec((2 * half, d_out), lambda i: (0, 0)),
        ],
        out_specs=pl.BlockSpec((_BR, d_out), lambda i: (i, 0)),
        out_shape=jax.ShapeDtypeStruct((n, d_out), jnp.float32),
    )(ga, gb, sa, sb, dinv, b1, w2)


def _final_body(gp_ref, s_ref, dinv_ref, b_ref, o_ref):
    dinv = dinv_ref[...]
    g = gp_ref[0] + gp_ref[1]
    o_ref[...] = dinv * (g + s_ref[...]) + b_ref[...]


def _final_call(g_part, s2, dinv, b2):
    _, n, d_out = g_part.shape
    return pl.pallas_call(
        _final_body,
        grid=(n // _BR,),
        in_specs=[
            pl.BlockSpec((NC, _BR, d_out), lambda i: (0, i, 0)),
            pl.BlockSpec((_BR, d_out), lambda i: (i, 0)),
            pl.BlockSpec((_BR, 1), lambda i: (i, 0)),
            pl.BlockSpec((1, d_out), lambda i: (0, 0)),
        ],
        out_specs=pl.BlockSpec((_BR, d_out), lambda i: (i, 0)),
        out_shape=jax.ShapeDtypeStruct((n, d_out), jnp.float32),
    )(g_part, s2, dinv, b2)


@jax.jit
def kernel(X, A, W1, b1, W2, b2):
    n = X.shape[0]
    e = A.shape[1]
    src = A[0]
    dst = A[1]
    src3 = src.reshape(NS, e // NS // CHUNK, CHUNK)
    dst3 = dst.reshape(NS, e // NS // CHUNK, CHUNK)
    src4 = src.reshape(NC * NS, e // (NC * NS) // CHUNK, CHUNK)
    dst4 = dst.reshape(NC * NS, e // (NC * NS) // CHUNK, CHUNK)

    deg_out = _deg_call(dst4, n)
    s1a, s1b, dinv = _mm1_call(X, W1, deg_out)
    g1a, g1b = _agg_call(True, s1a, s1b, src3, dst3)
    s2 = _mm2_call(g1a, g1b, s1a, s1b, dinv, b1.reshape(1, -1), W2)
    g2_part = _agg_call(False, s2, s2, src4, dst4)
    return _final_call(g2_part, s2, dinv, b2.reshape(1, -1))
